# baseline (speedup 1.0000x reference)
import functools

import jax
import jax.numpy as jnp
from jax import lax
from jax.experimental import pallas as pl
from jax.experimental.pallas import tpu as pltpu
from jax.experimental.pallas import tpu_sc as plsc

W_NUM = 100000
E_NUM = 1000000
D = 64
L2 = 1e-06
B = 16384
QL = 20
K = 5

NC, NS = 2, 16
NWK = NC * NS
CH = 512


def _sc_gather_all(entity_emb, word_emb, bias16,
                   users, items, negi, qwf, rw, nrw, rwb, nrwb):
    mesh = plsc.VectorSubcoreMesh(core_axis_name="c", subcore_axis_name="s",
                                  num_cores=NC, num_subcores=NS)
    out_type = (
        jax.ShapeDtypeStruct((B, D), jnp.float32),
        jax.ShapeDtypeStruct((B, D), jnp.float32),
        jax.ShapeDtypeStruct((B * K, D), jnp.float32),
        jax.ShapeDtypeStruct((B, D), jnp.float32),
        jax.ShapeDtypeStruct((B, D), jnp.float32),
        jax.ShapeDtypeStruct((B * K, D), jnp.float32),
        jax.ShapeDtypeStruct((B, 16), jnp.float32),
        jax.ShapeDtypeStruct((B * K, 16), jnp.float32),
    )
    QC = 32
    QROWS = QC * QL

    @functools.partial(
        pl.kernel, mesh=mesh, out_type=out_type,
        compiler_params=pltpu.CompilerParams(use_tc_tiling_on_sc=False),
        scratch_types=[
            pltpu.VMEM((CH,), jnp.int32),
            pltpu.VMEM((CH, D), jnp.float32),
            pltpu.VMEM((CH, 16), jnp.float32),
            pltpu.VMEM((QROWS,), jnp.int32),
            pltpu.VMEM((QROWS, D), jnp.float32),
            pltpu.VMEM((QC, D), jnp.float32),
            pltpu.SemaphoreType.DMA,
        ],
    )
    def k(ent_h, wrd_h, b16_h,
          i_users, i_items, i_negi, i_qw, i_rw, i_nrw, i_rwb, i_nrwb,
          o_user, o_item, o_nie, o_qsum, o_we, o_nwe, o_wb, o_nwb,
          idx_v, rows_v, brows_v, qidx_v, qrows_v, qsum_v, sem):
        wid = lax.axis_index("s") * NC + lax.axis_index("c")
        groups = [
            (ent_h, i_users, o_user, rows_v),
            (ent_h, i_items, o_item, rows_v),
            (ent_h, i_negi, o_nie, rows_v),
            (wrd_h, i_rw, o_we, rows_v),
            (wrd_h, i_nrw, o_nwe, rows_v),
            (b16_h, i_rwb, o_wb, brows_v),
            (b16_h, i_nrwb, o_nwb, brows_v),
        ]
        for tab, idxa, outa, rv in groups:
            n_w = idxa.shape[0] // NWK
            nch = n_w // CH
            base = wid * n_w

            def chunk(c, carry, tab=tab, idxa=idxa, outa=outa, rv=rv,
                      base=base):
                off = base + c * CH
                pltpu.sync_copy(idxa.at[pl.ds(off, CH)], idx_v)
                pltpu.async_copy(tab.at[idx_v], rv, sem).wait()
                pltpu.sync_copy(rv, outa.at[pl.ds(off, CH)])
                return carry

            lax.fori_loop(0, nch, chunk, 0)

        spw = B // NWK
        sbase = wid * spw

        def qchunk(c, carry):
            soff = sbase + c * QC
            pltpu.sync_copy(i_qw.at[pl.ds(soff * QL, QROWS)], qidx_v)
            pltpu.async_copy(wrd_h.at[qidx_v], qrows_v, sem).wait()

            def sample(s, carry2):
                accs = [jnp.zeros((16,), jnp.float32) for _ in range(D // 16)]
                for j in range(QL):
                    for v in range(D // 16):
                        accs[v] = accs[v] + qrows_v[s * QL + j,
                                                    pl.ds(v * 16, 16)]
                for v in range(D // 16):
                    qsum_v[s, pl.ds(v * 16, 16)] = accs[v]
                return carry2

            lax.fori_loop(0, QC, sample, 0)
            pltpu.sync_copy(qsum_v, o_qsum.at[pl.ds(soff, QC)])
            return carry

        lax.fori_loop(0, spw // QC, qchunk, 0)

    return k(entity_emb, word_emb, bias16,
             users, items, negi, qwf, rw, nrw, rwb, nrwb)


def _norms_sumsq(word_flat, entity_flat):
    G = 125

    def body(w_ref, e_ref, sw_ref, se_ref):
        i = pl.program_id(0)

        @pl.when(i == 0)
        def _():
            sw_ref[...] = jnp.zeros_like(sw_ref)
            se_ref[...] = jnp.zeros_like(se_ref)

        sw_ref[...] += jnp.sum(w_ref[...] * w_ref[...]).reshape(1, 1)
        se_ref[...] += jnp.sum(e_ref[...] * e_ref[...]).reshape(1, 1)

    return pl.pallas_call(
        body,
        grid=(G,),
        in_specs=[
            pl.BlockSpec((W_NUM * D // G // 1024, 8, 128),
                         lambda i: (i, 0, 0)),
            pl.BlockSpec((E_NUM * D // G // 1024, 8, 128),
                         lambda i: (i, 0, 0)),
        ],
        out_specs=[pl.BlockSpec((1, 1), lambda i: (0, 0))] * 2,
        out_shape=[jax.ShapeDtypeStruct((1, 1), jnp.float32)] * 2,
    )(word_flat.reshape(W_NUM * D // 1024, 8, 128),
      entity_flat.reshape(E_NUM * D // 1024, 8, 128))


def _log_sigmoid(x):
    return jnp.minimum(x, 0.0) - jnp.log1p(jnp.exp(-jnp.abs(x)))


def _loss_partials(user_e, item_e, qsum, nie, w_e, nw_e, wb16, nwb16,
                   rwmod, nrwmod, WqT, bq, pf):
    G = 32
    S = B // G

    def body(u_ref, it_ref, q_ref, nie_ref, we_ref, nwe_ref, wb_ref, nwb_ref,
             rwm_ref, nrwm_ref, wqt_ref, bq_ref, pf_ref,
             s1_ref, s2_ref, ns_ref, nw_ref, wbo_ref):
        i = pl.program_id(0)
        u = u_ref[...]
        it = it_ref[...]
        qmean = q_ref[...] * (1.0 / QL)
        q = jnp.tanh(jnp.dot(qmean, wqt_ref[...],
                             preferred_element_type=jnp.float32) + bq_ref[...])
        pf = pf_ref[0, 0]
        pm = pf * q + (1.0 - pf) * u

        s1p = jnp.sum(it * pm)
        nid = jnp.sum(nie_ref[...].reshape(S, K, D) * pm[:, None, :], axis=2)
        nsp = jnp.sum(-_log_sigmoid(-nid))

        s2p = jnp.sum(we_ref[...] * it)
        nwd = jnp.sum(nwe_ref[...].reshape(S, K, D) * it[:, None, :], axis=2)

        lanes = lax.broadcasted_iota(jnp.int32, (S * K, 16), 1)
        nwb = jnp.sum(jnp.where(lanes == nrwm_ref[0, 0, :][:, None],
                                nwb_ref[...], 0.0), axis=1).reshape(S, K)
        nwp = jnp.sum(-_log_sigmoid(-nwd - nwb))

        lanes2 = lax.broadcasted_iota(jnp.int32, (S, 16), 1)
        wb = jnp.sum(jnp.where(lanes2 == rwm_ref[0, 0, :][:, None],
                               wb_ref[...], 0.0), axis=1)
        wbo_ref[...] = wb.reshape(1, 1, S)

        @pl.when(i == 0)
        def _():
            s1_ref[...] = jnp.zeros_like(s1_ref)
            s2_ref[...] = jnp.zeros_like(s2_ref)
            ns_ref[...] = jnp.zeros_like(ns_ref)
            nw_ref[...] = jnp.zeros_like(nw_ref)

        s1_ref[...] += s1p.reshape(1, 1)
        s2_ref[...] += s2p.reshape(1, 1)
        ns_ref[...] += nsp.reshape(1, 1)
        nw_ref[...] += nwp.reshape(1, 1)

    return pl.pallas_call(
        body,
        grid=(G,),
        in_specs=[
            pl.BlockSpec((S, D), lambda i: (i, 0)),
            pl.BlockSpec((S, D), lambda i: (i, 0)),
            pl.BlockSpec((S, D), lambda i: (i, 0)),
            pl.BlockSpec((S * K, D), lambda i: (i, 0)),
            pl.BlockSpec((S, D), lambda i: (i, 0)),
            pl.BlockSpec((S * K, D), lambda i: (i, 0)),
            pl.BlockSpec((S, 16), lambda i: (i, 0)),
            pl.BlockSpec((S * K, 16), lambda i: (i, 0)),
            pl.BlockSpec((1, 1, S), lambda i: (i, 0, 0)),
            pl.BlockSpec((1, 1, S * K), lambda i: (i, 0, 0)),
            pl.BlockSpec((D, D), lambda i: (0, 0)),
            pl.BlockSpec((1, D), lambda i: (0, 0)),
            pl.BlockSpec((1, 1), lambda i: (0, 0)),
        ],
        out_specs=[
            pl.BlockSpec((1, 1), lambda i: (0, 0)),
            pl.BlockSpec((1, 1), lambda i: (0, 0)),
            pl.BlockSpec((1, 1), lambda i: (0, 0)),
            pl.BlockSpec((1, 1), lambda i: (0, 0)),
            pl.BlockSpec((1, 1, S), lambda i: (i, 0, 0)),
        ],
        out_shape=[
            jax.ShapeDtypeStruct((1, 1), jnp.float32),
            jax.ShapeDtypeStruct((1, 1), jnp.float32),
            jax.ShapeDtypeStruct((1, 1), jnp.float32),
            jax.ShapeDtypeStruct((1, 1), jnp.float32),
            jax.ShapeDtypeStruct((G, 1, S), jnp.float32),
        ],
    )(user_e, item_e, qsum, nie, w_e, nw_e, wb16, nwb16, rwmod, nrwmod,
      WqT, bq, pf)


def _final_combine(s1, s2, ns, nw, ssw, sse, wb2d):
    def body(s1_ref, s2_ref, ns_ref, nw_ref, ssw_ref, sse_ref, wb_ref, o_ref):
        s2 = s2_ref[0, 0]
        pos_mean = jnp.mean(-_log_sigmoid(s2 + wb_ref[...]))
        search = -_log_sigmoid(s1_ref[0, 0]) + ns_ref[0, 0]
        reg = L2 * (jnp.sqrt(ssw_ref[0, 0]) + jnp.sqrt(sse_ref[0, 0]))
        o_ref[...] = (pos_mean + nw_ref[0, 0] / B + search + reg).reshape(1, 1)

    return pl.pallas_call(
        body,
        out_shape=jax.ShapeDtypeStruct((1, 1), jnp.float32),
    )(s1, s2, ns, nw, ssw, sse, wb2d)


def kernel(users, items, query_words, review_words, neg_items,
           neg_review_words, word_emb, word_bias, entity_emb, Wq, bq, pf):
    users = users.astype(jnp.int32)
    items = items.astype(jnp.int32)
    qwf = query_words.astype(jnp.int32).reshape(-1)
    rw = review_words.astype(jnp.int32)
    nrw = neg_review_words.astype(jnp.int32).reshape(-1)
    negi = neg_items.astype(jnp.int32).reshape(-1)

    bias16 = word_bias.reshape(W_NUM // 16, 16)
    rwb = rw // 16
    nrwb = nrw // 16
    rwmod = (rw % 16).reshape(32, 1, B // 32)
    nrwmod = (nrw % 16).reshape(32, 1, (B * K) // 32)

    (user_e, item_e, nie, qsum, w_e, nw_e, wb16, nwb16) = _sc_gather_all(
        entity_emb, word_emb, bias16, users, items, negi, qwf, rw, nrw,
        rwb, nrwb)

    ssw, sse = _norms_sumsq(word_emb.T.reshape(-1), entity_emb.T.reshape(-1))

    s1, s2, ns, nw, wbo = _loss_partials(
        user_e, item_e, qsum, nie, w_e, nw_e, wb16, nwb16, rwmod, nrwmod,
        Wq.T, bq.reshape(1, D), pf.reshape(1, 1))

    out = _final_combine(s1, s2, ns, nw, ssw, sse, wbo.reshape(128, 128))
    return out.reshape(())


# speedup vs baseline: 5.3157x; 5.3157x over previous
import functools

import jax
import jax.numpy as jnp
from jax import lax
from jax.experimental import pallas as pl
from jax.experimental.pallas import tpu as pltpu
from jax.experimental.pallas import tpu_sc as plsc

W_NUM = 100000
E_NUM = 1000000
D = 64
L2 = 1e-06
B = 16384
QL = 20
K = 5

NC, NS = 2, 16
NWK = NC * NS
CH = 512


def _sc_gather_all(entity_emb, word_emb, bias16,
                   users, items, negi, qwf, rw, nrw, rwb, nrwb):
    mesh = plsc.VectorSubcoreMesh(core_axis_name="c", subcore_axis_name="s",
                                  num_cores=NC, num_subcores=NS)
    out_type = (
        jax.ShapeDtypeStruct((B, D), jnp.float32),
        jax.ShapeDtypeStruct((B, D), jnp.float32),
        jax.ShapeDtypeStruct((B * K, D), jnp.float32),
        jax.ShapeDtypeStruct((B, D), jnp.float32),
        jax.ShapeDtypeStruct((B, D), jnp.float32),
        jax.ShapeDtypeStruct((B * K, D), jnp.float32),
        jax.ShapeDtypeStruct((B, 16), jnp.float32),
        jax.ShapeDtypeStruct((B * K, 16), jnp.float32),
    )
    QC = 32
    QROWS = QC * QL

    @functools.partial(
        pl.kernel, mesh=mesh, out_type=out_type,
        compiler_params=pltpu.CompilerParams(use_tc_tiling_on_sc=False),
        scratch_types=[
            pltpu.VMEM((CH,), jnp.int32),
            pltpu.VMEM((CH, D), jnp.float32),
            pltpu.VMEM((CH, 16), jnp.float32),
            pltpu.VMEM((QROWS,), jnp.int32),
            pltpu.VMEM((QROWS, D), jnp.float32),
            pltpu.VMEM((QC, D), jnp.float32),
            pltpu.SemaphoreType.DMA,
        ],
    )
    def k(ent_h, wrd_h, b16_h,
          i_users, i_items, i_negi, i_qw, i_rw, i_nrw, i_rwb, i_nrwb,
          o_user, o_item, o_nie, o_qsum, o_we, o_nwe, o_wb, o_nwb,
          idx_v, rows_v, brows_v, qidx_v, qrows_v, qsum_v, sem):
        wid = lax.axis_index("s") * NC + lax.axis_index("c")
        groups = [
            (ent_h, i_users, o_user, rows_v),
            (ent_h, i_items, o_item, rows_v),
            (ent_h, i_negi, o_nie, rows_v),
            (wrd_h, i_rw, o_we, rows_v),
            (wrd_h, i_nrw, o_nwe, rows_v),
            (b16_h, i_rwb, o_wb, brows_v),
            (b16_h, i_nrwb, o_nwb, brows_v),
        ]
        for tab, idxa, outa, rv in groups:
            n_w = idxa.shape[0] // NWK
            nch = n_w // CH
            base = wid * n_w

            def chunk(c, carry, tab=tab, idxa=idxa, outa=outa, rv=rv,
                      base=base):
                off = base + c * CH
                pltpu.sync_copy(idxa.at[pl.ds(off, CH)], idx_v)
                pltpu.async_copy(tab.at[idx_v], rv, sem).wait()
                pltpu.sync_copy(rv, outa.at[pl.ds(off, CH)])
                return carry

            lax.fori_loop(0, nch, chunk, 0)

        spw = B // NWK
        sbase = wid * spw

        def qchunk(c, carry):
            soff = sbase + c * QC
            pltpu.sync_copy(i_qw.at[pl.ds(soff * QL, QROWS)], qidx_v)
            pltpu.async_copy(wrd_h.at[qidx_v], qrows_v, sem).wait()

            def sample(s, carry2):
                accs = [jnp.zeros((16,), jnp.float32) for _ in range(D // 16)]
                for j in range(QL):
                    for v in range(D // 16):
                        accs[v] = accs[v] + qrows_v[s * QL + j,
                                                    pl.ds(v * 16, 16)]
                for v in range(D // 16):
                    qsum_v[s, pl.ds(v * 16, 16)] = accs[v]
                return carry2

            lax.fori_loop(0, QC, sample, 0)
            pltpu.sync_copy(qsum_v, o_qsum.at[pl.ds(soff, QC)])
            return carry

        lax.fori_loop(0, spw // QC, qchunk, 0)

    return k(entity_emb, word_emb, bias16,
             users, items, negi, qwf, rw, nrw, rwb, nrwb)


def _norms_sumsq(word_emb, entity_emb):
    G = 125

    def body(w_ref, e_ref, sw_ref, se_ref):
        i = pl.program_id(0)

        @pl.when(i == 0)
        def _():
            sw_ref[...] = jnp.zeros_like(sw_ref)
            se_ref[...] = jnp.zeros_like(se_ref)

        sw_ref[...] += jnp.sum(w_ref[...] * w_ref[...]).reshape(1, 1)
        se_ref[...] += jnp.sum(e_ref[...] * e_ref[...]).reshape(1, 1)

    return pl.pallas_call(
        body,
        grid=(G,),
        in_specs=[
            pl.BlockSpec((W_NUM // G, D), lambda i: (i, 0)),
            pl.BlockSpec((E_NUM // G, D), lambda i: (i, 0)),
        ],
        out_specs=[pl.BlockSpec((1, 1), lambda i: (0, 0))] * 2,
        out_shape=[jax.ShapeDtypeStruct((1, 1), jnp.float32)] * 2,
    )(word_emb, entity_emb)


def _log_sigmoid(x):
    return jnp.minimum(x, 0.0) - jnp.log1p(jnp.exp(-jnp.abs(x)))


def _loss_partials(user_e, item_e, qsum, nie, w_e, nw_e, wb16, nwb16,
                   rwmod, nrwmod, WqT, bq, pf):
    G = 32
    S = B // G

    def body(u_ref, it_ref, q_ref, nie_ref, we_ref, nwe_ref, wb_ref, nwb_ref,
             rwm_ref, nrwm_ref, wqt_ref, bq_ref, pf_ref,
             s1_ref, s2_ref, ns_ref, nw_ref, wbo_ref):
        i = pl.program_id(0)
        u = u_ref[...]
        it = it_ref[...]
        qmean = q_ref[...] * (1.0 / QL)
        q = jnp.tanh(jnp.dot(qmean, wqt_ref[...],
                             preferred_element_type=jnp.float32) + bq_ref[...])
        pf = pf_ref[0, 0]
        pm = pf * q + (1.0 - pf) * u

        s1p = jnp.sum(it * pm)
        nid = jnp.sum(nie_ref[...].reshape(S, K, D) * pm[:, None, :], axis=2)
        nsp = jnp.sum(-_log_sigmoid(-nid))

        s2p = jnp.sum(we_ref[...] * it)
        nwd = jnp.sum(nwe_ref[...].reshape(S, K, D) * it[:, None, :], axis=2)

        lanes = lax.broadcasted_iota(jnp.int32, (S * K, 16), 1)
        nwb = jnp.sum(jnp.where(lanes == nrwm_ref[0, 0, :][:, None],
                                nwb_ref[...], 0.0), axis=1).reshape(S, K)
        nwp = jnp.sum(-_log_sigmoid(-nwd - nwb))

        lanes2 = lax.broadcasted_iota(jnp.int32, (S, 16), 1)
        wb = jnp.sum(jnp.where(lanes2 == rwm_ref[0, 0, :][:, None],
                               wb_ref[...], 0.0), axis=1)
        wbo_ref[...] = wb.reshape(1, 1, S)

        @pl.when(i == 0)
        def _():
            s1_ref[...] = jnp.zeros_like(s1_ref)
            s2_ref[...] = jnp.zeros_like(s2_ref)
            ns_ref[...] = jnp.zeros_like(ns_ref)
            nw_ref[...] = jnp.zeros_like(nw_ref)

        s1_ref[...] += s1p.reshape(1, 1)
        s2_ref[...] += s2p.reshape(1, 1)
        ns_ref[...] += nsp.reshape(1, 1)
        nw_ref[...] += nwp.reshape(1, 1)

    return pl.pallas_call(
        body,
        grid=(G,),
        in_specs=[
            pl.BlockSpec((S, D), lambda i: (i, 0)),
            pl.BlockSpec((S, D), lambda i: (i, 0)),
            pl.BlockSpec((S, D), lambda i: (i, 0)),
            pl.BlockSpec((S * K, D), lambda i: (i, 0)),
            pl.BlockSpec((S, D), lambda i: (i, 0)),
            pl.BlockSpec((S * K, D), lambda i: (i, 0)),
            pl.BlockSpec((S, 16), lambda i: (i, 0)),
            pl.BlockSpec((S * K, 16), lambda i: (i, 0)),
            pl.BlockSpec((1, 1, S), lambda i: (i, 0, 0)),
            pl.BlockSpec((1, 1, S * K), lambda i: (i, 0, 0)),
            pl.BlockSpec((D, D), lambda i: (0, 0)),
            pl.BlockSpec((1, D), lambda i: (0, 0)),
            pl.BlockSpec((1, 1), lambda i: (0, 0)),
        ],
        out_specs=[
            pl.BlockSpec((1, 1), lambda i: (0, 0)),
            pl.BlockSpec((1, 1), lambda i: (0, 0)),
            pl.BlockSpec((1, 1), lambda i: (0, 0)),
            pl.BlockSpec((1, 1), lambda i: (0, 0)),
            pl.BlockSpec((1, 1, S), lambda i: (i, 0, 0)),
        ],
        out_shape=[
            jax.ShapeDtypeStruct((1, 1), jnp.float32),
            jax.ShapeDtypeStruct((1, 1), jnp.float32),
            jax.ShapeDtypeStruct((1, 1), jnp.float32),
            jax.ShapeDtypeStruct((1, 1), jnp.float32),
            jax.ShapeDtypeStruct((G, 1, S), jnp.float32),
        ],
    )(user_e, item_e, qsum, nie, w_e, nw_e, wb16, nwb16, rwmod, nrwmod,
      WqT, bq, pf)


def _final_combine(s1, s2, ns, nw, ssw, sse, wb2d):
    def body(s1_ref, s2_ref, ns_ref, nw_ref, ssw_ref, sse_ref, wb_ref, o_ref):
        s2 = s2_ref[0, 0]
        pos_mean = jnp.mean(-_log_sigmoid(s2 + wb_ref[...]))
        search = -_log_sigmoid(s1_ref[0, 0]) + ns_ref[0, 0]
        reg = L2 * (jnp.sqrt(ssw_ref[0, 0]) + jnp.sqrt(sse_ref[0, 0]))
        o_ref[...] = (pos_mean + nw_ref[0, 0] / B + search + reg).reshape(1, 1)

    return pl.pallas_call(
        body,
        out_shape=jax.ShapeDtypeStruct((1, 1), jnp.float32),
    )(s1, s2, ns, nw, ssw, sse, wb2d)


def kernel(users, items, query_words, review_words, neg_items,
           neg_review_words, word_emb, word_bias, entity_emb, Wq, bq, pf):
    users = users.astype(jnp.int32)
    items = items.astype(jnp.int32)
    qwf = query_words.astype(jnp.int32).reshape(-1)
    rw = review_words.astype(jnp.int32)
    nrw = neg_review_words.astype(jnp.int32).reshape(-1)
    negi = neg_items.astype(jnp.int32).reshape(-1)

    bias16 = word_bias.reshape(W_NUM // 16, 16)
    rwb = rw // 16
    nrwb = nrw // 16
    rwmod = (rw % 16).reshape(32, 1, B // 32)
    nrwmod = (nrw % 16).reshape(32, 1, (B * K) // 32)

    (user_e, item_e, nie, qsum, w_e, nw_e, wb16, nwb16) = _sc_gather_all(
        entity_emb, word_emb, bias16, users, items, negi, qwf, rw, nrw,
        rwb, nrwb)

    ssw, sse = _norms_sumsq(word_emb, entity_emb)

    s1, s2, ns, nw, wbo = _loss_partials(
        user_e, item_e, qsum, nie, w_e, nw_e, wb16, nwb16, rwmod, nrwmod,
        Wq.T, bq.reshape(1, D), pf.reshape(1, 1))

    out = _final_combine(s1, s2, ns, nw, ssw, sse, wbo.reshape(128, 128))
    return out.reshape(())


# speedup vs baseline: 5.6327x; 1.0596x over previous
import functools

import jax
import jax.numpy as jnp
from jax import lax
from jax.experimental import pallas as pl
from jax.experimental.pallas import tpu as pltpu
from jax.experimental.pallas import tpu_sc as plsc

W_NUM = 100000
E_NUM = 1000000
D = 64
L2 = 1e-06
B = 16384
QL = 20
K = 5

NC, NS = 2, 16
NWK = NC * NS
CH = 512


def _sc_gather_all(entity_emb, word_emb, bias16,
                   users, items, negi, qwf, rw, nrw, rwb, nrwb):
    mesh = plsc.VectorSubcoreMesh(core_axis_name="c", subcore_axis_name="s",
                                  num_cores=NC, num_subcores=NS)
    out_type = (
        jax.ShapeDtypeStruct((B, D), jnp.float32),
        jax.ShapeDtypeStruct((B, D), jnp.float32),
        jax.ShapeDtypeStruct((B * K, D), jnp.float32),
        jax.ShapeDtypeStruct((B, D), jnp.float32),
        jax.ShapeDtypeStruct((B, D), jnp.float32),
        jax.ShapeDtypeStruct((B * K, D), jnp.float32),
        jax.ShapeDtypeStruct((B, 16), jnp.float32),
        jax.ShapeDtypeStruct((B * K, 16), jnp.float32),
        jax.ShapeDtypeStruct((NWK, 16), jnp.float32),
        jax.ShapeDtypeStruct((NWK, 16), jnp.float32),
    )
    QC = 32
    QROWS = QC * QL
    NCH = 250

    @functools.partial(
        pl.kernel, mesh=mesh, out_type=out_type,
        compiler_params=pltpu.CompilerParams(use_tc_tiling_on_sc=False),
        scratch_types=[
            pltpu.VMEM((CH,), jnp.int32),
            pltpu.VMEM((CH, D), jnp.float32),
            pltpu.VMEM((CH, 16), jnp.float32),
            pltpu.VMEM((QROWS,), jnp.int32),
            pltpu.VMEM((QROWS, D), jnp.float32),
            pltpu.VMEM((QC, D), jnp.float32),
            pltpu.VMEM((NCH, D), jnp.float32),
            pltpu.VMEM((NCH, D), jnp.float32),
            pltpu.VMEM((16,), jnp.float32),
            pltpu.SemaphoreType.DMA,
            pltpu.SemaphoreType.DMA,
            pltpu.SemaphoreType.DMA,
        ],
    )
    def k(ent_h, wrd_h, b16_h,
          i_users, i_items, i_negi, i_qw, i_rw, i_nrw, i_rwb, i_nrwb,
          o_user, o_item, o_nie, o_qsum, o_we, o_nwe, o_wb, o_nwb,
          o_wssq, o_essq,
          idx_v, rows_v, brows_v, qidx_v, qrows_v, qsum_v,
          nbuf0, nbuf1, nacc_v, sem, nsem0, nsem1):
        wid = lax.axis_index("s") * NC + lax.axis_index("c")
        groups = [
            (ent_h, i_users, o_user, rows_v),
            (ent_h, i_items, o_item, rows_v),
            (ent_h, i_negi, o_nie, rows_v),
            (wrd_h, i_rw, o_we, rows_v),
            (wrd_h, i_nrw, o_nwe, rows_v),
            (b16_h, i_rwb, o_wb, brows_v),
            (b16_h, i_nrwb, o_nwb, brows_v),
        ]
        for tab, idxa, outa, rv in groups:
            n_w = idxa.shape[0] // NWK
            nch = n_w // CH
            base = wid * n_w

            def chunk(c, carry, tab=tab, idxa=idxa, outa=outa, rv=rv,
                      base=base):
                off = base + c * CH
                pltpu.sync_copy(idxa.at[pl.ds(off, CH)], idx_v)
                pltpu.async_copy(tab.at[idx_v], rv, sem).wait()
                pltpu.sync_copy(rv, outa.at[pl.ds(off, CH)])
                return carry

            lax.fori_loop(0, nch, chunk, 0)

        spw = B // NWK
        sbase = wid * spw

        def qchunk(c, carry):
            soff = sbase + c * QC
            pltpu.sync_copy(i_qw.at[pl.ds(soff * QL, QROWS)], qidx_v)
            pltpu.async_copy(wrd_h.at[qidx_v], qrows_v, sem).wait()

            def sample(s, carry2):
                accs = [jnp.zeros((16,), jnp.float32) for _ in range(D // 16)]
                for j in range(QL):
                    for v in range(D // 16):
                        accs[v] = accs[v] + qrows_v[s * QL + j,
                                                    pl.ds(v * 16, 16)]
                for v in range(D // 16):
                    qsum_v[s, pl.ds(v * 16, 16)] = accs[v]
                return carry2

            lax.fori_loop(0, QC, sample, 0)
            pltpu.sync_copy(qsum_v, o_qsum.at[pl.ds(soff, QC)])
            return carry

        lax.fori_loop(0, spw // QC, qchunk, 0)

        def table_ssq(tab, out_row, cn):
            rows = tab.shape[0] // NWK
            nch = rows // cn
            tbase = wid * rows
            zero4 = tuple(jnp.zeros((16,), jnp.float32) for _ in range(4))

            def reduce_buf(buf, accs):
                def row(r, a):
                    res = []
                    for v in range(D // 16):
                        x = buf[r, pl.ds(v * 16, 16)]
                        res.append(a[v] + x * x)
                    return tuple(res)
                return lax.fori_loop(0, cn, row, accs)

            def pair(p, accs):
                d0 = pltpu.async_copy(
                    tab.at[pl.ds(tbase + (2 * p) * cn, cn)],
                    nbuf0.at[pl.ds(0, cn)], nsem0)
                d1 = pltpu.async_copy(
                    tab.at[pl.ds(tbase + (2 * p + 1) * cn, cn)],
                    nbuf1.at[pl.ds(0, cn)], nsem1)
                d0.wait()
                accs = reduce_buf(nbuf0, accs)
                d1.wait()
                accs = reduce_buf(nbuf1, accs)
                return accs

            accs = lax.fori_loop(0, nch // 2, pair, zero4)
            if nch % 2:
                pltpu.sync_copy(tab.at[pl.ds(tbase + (nch - 1) * cn, cn)],
                                nbuf0.at[pl.ds(0, cn)])
                accs = reduce_buf(nbuf0, accs)
            nacc_v[...] = accs[0] + accs[1] + accs[2] + accs[3]
            pltpu.sync_copy(nacc_v, out_row)

        table_ssq(wrd_h, o_wssq.at[wid], 125)
        table_ssq(ent_h, o_essq.at[wid], NCH)

    return k(entity_emb, word_emb, bias16,
             users, items, negi, qwf, rw, nrw, rwb, nrwb)


def _log_sigmoid(x):
    return jnp.minimum(x, 0.0) - jnp.log1p(jnp.exp(-jnp.abs(x)))


def _loss_partials(user_e, item_e, qsum, nie, w_e, nw_e, wb16, nwb16,
                   rwmod, nrwmod, WqT, bq, pf):
    G = 32
    S = B // G

    def body(u_ref, it_ref, q_ref, nie_ref, we_ref, nwe_ref, wb_ref, nwb_ref,
             rwm_ref, nrwm_ref, wqt_ref, bq_ref, pf_ref,
             s1_ref, s2_ref, ns_ref, nw_ref, wbo_ref):
        i = pl.program_id(0)
        u = u_ref[...]
        it = it_ref[...]
        qmean = q_ref[...] * (1.0 / QL)
        q = jnp.tanh(jnp.dot(qmean, wqt_ref[...],
                             preferred_element_type=jnp.float32) + bq_ref[...])
        pf = pf_ref[0, 0]
        pm = pf * q + (1.0 - pf) * u

        s1p = jnp.sum(it * pm)
        nid = jnp.sum(nie_ref[...].reshape(S, K, D) * pm[:, None, :], axis=2)
        nsp = jnp.sum(-_log_sigmoid(-nid))

        s2p = jnp.sum(we_ref[...] * it)
        nwd = jnp.sum(nwe_ref[...].reshape(S, K, D) * it[:, None, :], axis=2)

        lanes = lax.broadcasted_iota(jnp.int32, (S * K, 16), 1)
        nwb = jnp.sum(jnp.where(lanes == nrwm_ref[0, 0, :][:, None],
                                nwb_ref[...], 0.0), axis=1).reshape(S, K)
        nwp = jnp.sum(-_log_sigmoid(-nwd - nwb))

        lanes2 = lax.broadcasted_iota(jnp.int32, (S, 16), 1)
        wb = jnp.sum(jnp.where(lanes2 == rwm_ref[0, 0, :][:, None],
                               wb_ref[...], 0.0), axis=1)
        wbo_ref[...] = wb.reshape(1, 1, S)

        @pl.when(i == 0)
        def _():
            s1_ref[...] = jnp.zeros_like(s1_ref)
            s2_ref[...] = jnp.zeros_like(s2_ref)
            ns_ref[...] = jnp.zeros_like(ns_ref)
            nw_ref[...] = jnp.zeros_like(nw_ref)

        s1_ref[...] += s1p.reshape(1, 1)
        s2_ref[...] += s2p.reshape(1, 1)
        ns_ref[...] += nsp.reshape(1, 1)
        nw_ref[...] += nwp.reshape(1, 1)

    return pl.pallas_call(
        body,
        grid=(G,),
        in_specs=[
            pl.BlockSpec((S, D), lambda i: (i, 0)),
            pl.BlockSpec((S, D), lambda i: (i, 0)),
            pl.BlockSpec((S, D), lambda i: (i, 0)),
            pl.BlockSpec((S * K, D), lambda i: (i, 0)),
            pl.BlockSpec((S, D), lambda i: (i, 0)),
            pl.BlockSpec((S * K, D), lambda i: (i, 0)),
            pl.BlockSpec((S, 16), lambda i: (i, 0)),
            pl.BlockSpec((S * K, 16), lambda i: (i, 0)),
            pl.BlockSpec((1, 1, S), lambda i: (i, 0, 0)),
            pl.BlockSpec((1, 1, S * K), lambda i: (i, 0, 0)),
            pl.BlockSpec((D, D), lambda i: (0, 0)),
            pl.BlockSpec((1, D), lambda i: (0, 0)),
            pl.BlockSpec((1, 1), lambda i: (0, 0)),
        ],
        out_specs=[
            pl.BlockSpec((1, 1), lambda i: (0, 0)),
            pl.BlockSpec((1, 1), lambda i: (0, 0)),
            pl.BlockSpec((1, 1), lambda i: (0, 0)),
            pl.BlockSpec((1, 1), lambda i: (0, 0)),
            pl.BlockSpec((1, 1, S), lambda i: (i, 0, 0)),
        ],
        out_shape=[
            jax.ShapeDtypeStruct((1, 1), jnp.float32),
            jax.ShapeDtypeStruct((1, 1), jnp.float32),
            jax.ShapeDtypeStruct((1, 1), jnp.float32),
            jax.ShapeDtypeStruct((1, 1), jnp.float32),
            jax.ShapeDtypeStruct((G, 1, S), jnp.float32),
        ],
    )(user_e, item_e, qsum, nie, w_e, nw_e, wb16, nwb16, rwmod, nrwmod,
      WqT, bq, pf)


def _final_combine(s1, s2, ns, nw, wssq, essq, wb2d):
    def body(s1_ref, s2_ref, ns_ref, nw_ref, ssw_ref, sse_ref, wb_ref, o_ref):
        s2 = s2_ref[0, 0]
        pos_mean = jnp.mean(-_log_sigmoid(s2 + wb_ref[...]))
        search = -_log_sigmoid(s1_ref[0, 0]) + ns_ref[0, 0]
        reg = L2 * (jnp.sqrt(jnp.sum(ssw_ref[...])) +
                    jnp.sqrt(jnp.sum(sse_ref[...])))
        o_ref[...] = (pos_mean + nw_ref[0, 0] / B + search + reg).reshape(1, 1)

    return pl.pallas_call(
        body,
        out_shape=jax.ShapeDtypeStruct((1, 1), jnp.float32),
    )(s1, s2, ns, nw, wssq, essq, wb2d)


def kernel(users, items, query_words, review_words, neg_items,
           neg_review_words, word_emb, word_bias, entity_emb, Wq, bq, pf):
    users = users.astype(jnp.int32)
    items = items.astype(jnp.int32)
    qwf = query_words.astype(jnp.int32).reshape(-1)
    rw = review_words.astype(jnp.int32)
    nrw = neg_review_words.astype(jnp.int32).reshape(-1)
    negi = neg_items.astype(jnp.int32).reshape(-1)

    bias16 = word_bias.reshape(W_NUM // 16, 16)
    rwb = rw // 16
    nrwb = nrw // 16
    rwmod = (rw % 16).reshape(32, 1, B // 32)
    nrwmod = (nrw % 16).reshape(32, 1, (B * K) // 32)

    (user_e, item_e, nie, qsum, w_e, nw_e, wb16, nwb16, wssq, essq) = (
        _sc_gather_all(entity_emb, word_emb, bias16, users, items, negi,
                       qwf, rw, nrw, rwb, nrwb))

    s1, s2, ns, nw, wbo = _loss_partials(
        user_e, item_e, qsum, nie, w_e, nw_e, wb16, nwb16, rwmod, nrwmod,
        Wq.T, bq.reshape(1, D), pf.reshape(1, 1))

    out = _final_combine(s1, s2, ns, nw, wssq, essq, wbo.reshape(128, 128))
    return out.reshape(())


# speedup vs baseline: 5.9940x; 1.0642x over previous
import functools

import jax
import jax.numpy as jnp
from jax import lax
from jax.experimental import pallas as pl
from jax.experimental.pallas import tpu as pltpu
from jax.experimental.pallas import tpu_sc as plsc

W_NUM = 100000
E_NUM = 1000000
D = 64
L2 = 1e-06
B = 16384
QL = 20
K = 5

NC, NS = 2, 16
NWK = NC * NS
CH = 512


def _sc_gather_all(entity_emb, word_emb, bias16,
                   users, items, negi, qwf, rw, nrw, rwb, nrwb):
    mesh = plsc.VectorSubcoreMesh(core_axis_name="c", subcore_axis_name="s",
                                  num_cores=NC, num_subcores=NS)
    out_type = (
        jax.ShapeDtypeStruct((B, D), jnp.float32),
        jax.ShapeDtypeStruct((B, D), jnp.float32),
        jax.ShapeDtypeStruct((B * K, D), jnp.float32),
        jax.ShapeDtypeStruct((B, D), jnp.float32),
        jax.ShapeDtypeStruct((B, D), jnp.float32),
        jax.ShapeDtypeStruct((B * K, D), jnp.float32),
        jax.ShapeDtypeStruct((B, 16), jnp.float32),
        jax.ShapeDtypeStruct((B * K, 16), jnp.float32),
    )
    QC = 32
    QROWS = QC * QL

    @functools.partial(
        pl.kernel, mesh=mesh, out_type=out_type,
        compiler_params=pltpu.CompilerParams(use_tc_tiling_on_sc=False),
        scratch_types=[
            pltpu.VMEM((CH,), jnp.int32),
            pltpu.VMEM((CH, D), jnp.float32),
            pltpu.VMEM((CH, 16), jnp.float32),
            pltpu.VMEM((QROWS,), jnp.int32),
            pltpu.VMEM((QROWS, D), jnp.float32),
            pltpu.VMEM((QC, D), jnp.float32),
            pltpu.SemaphoreType.DMA,
        ],
    )
    def k(ent_h, wrd_h, b16_h,
          i_users, i_items, i_negi, i_qw, i_rw, i_nrw, i_rwb, i_nrwb,
          o_user, o_item, o_nie, o_qsum, o_we, o_nwe, o_wb, o_nwb,
          idx_v, rows_v, brows_v, qidx_v, qrows_v, qsum_v, sem):
        wid = lax.axis_index("s") * NC + lax.axis_index("c")
        groups = [
            (ent_h, i_users, o_user, rows_v),
            (ent_h, i_items, o_item, rows_v),
            (ent_h, i_negi, o_nie, rows_v),
            (wrd_h, i_rw, o_we, rows_v),
            (wrd_h, i_nrw, o_nwe, rows_v),
            (b16_h, i_rwb, o_wb, brows_v),
            (b16_h, i_nrwb, o_nwb, brows_v),
        ]
        for tab, idxa, outa, rv in groups:
            n_w = idxa.shape[0] // NWK
            nch = n_w // CH
            base = wid * n_w

            def chunk(c, carry, tab=tab, idxa=idxa, outa=outa, rv=rv,
                      base=base):
                off = base + c * CH
                pltpu.sync_copy(idxa.at[pl.ds(off, CH)], idx_v)
                pltpu.async_copy(tab.at[idx_v], rv, sem).wait()
                pltpu.sync_copy(rv, outa.at[pl.ds(off, CH)])
                return carry

            lax.fori_loop(0, nch, chunk, 0)

        spw = B // NWK
        sbase = wid * spw

        def qchunk(c, carry):
            soff = sbase + c * QC
            pltpu.sync_copy(i_qw.at[pl.ds(soff * QL, QROWS)], qidx_v)
            pltpu.async_copy(wrd_h.at[qidx_v], qrows_v, sem).wait()

            def sample(s, carry2):
                accs = [jnp.zeros((16,), jnp.float32) for _ in range(D // 16)]
                for j in range(QL):
                    for v in range(D // 16):
                        accs[v] = accs[v] + qrows_v[s * QL + j,
                                                    pl.ds(v * 16, 16)]
                for v in range(D // 16):
                    qsum_v[s, pl.ds(v * 16, 16)] = accs[v]
                return carry2

            lax.fori_loop(0, QC, sample, 0)
            pltpu.sync_copy(qsum_v, o_qsum.at[pl.ds(soff, QC)])
            return carry

        lax.fori_loop(0, spw // QC, qchunk, 0)

    return k(entity_emb, word_emb, bias16,
             users, items, negi, qwf, rw, nrw, rwb, nrwb)


def _sc_norms(entity_emb, word_emb):
    mesh = plsc.VectorSubcoreMesh(core_axis_name="c", subcore_axis_name="s",
                                  num_cores=NC, num_subcores=NS)
    out_type = (
        jax.ShapeDtypeStruct((NWK, 16), jnp.float32),
        jax.ShapeDtypeStruct((NWK, 16), jnp.float32),
    )
    NCH = 250

    @functools.partial(
        pl.kernel, mesh=mesh, out_type=out_type,
        compiler_params=pltpu.CompilerParams(use_tc_tiling_on_sc=False),
        scratch_types=[
            pltpu.VMEM((NCH, D), jnp.float32),
            pltpu.VMEM((NCH, D), jnp.float32),
            pltpu.VMEM((16,), jnp.float32),
            pltpu.SemaphoreType.DMA,
            pltpu.SemaphoreType.DMA,
        ],
    )
    def k(ent_h, wrd_h, o_wssq, o_essq, nbuf0, nbuf1, nacc_v, nsem0, nsem1):
        wid = lax.axis_index("s") * NC + lax.axis_index("c")

        def table_ssq(tab, out_row, cn):
            rows = tab.shape[0] // NWK
            nch = rows // cn
            tbase = wid * rows
            zero4 = tuple(jnp.zeros((16,), jnp.float32) for _ in range(4))

            def reduce_buf(buf, accs):
                def row(r, a):
                    res = []
                    for v in range(D // 16):
                        x = buf[r, pl.ds(v * 16, 16)]
                        res.append(a[v] + x * x)
                    return tuple(res)
                return lax.fori_loop(0, cn, row, accs)

            def pair(p, accs):
                d0 = pltpu.async_copy(
                    tab.at[pl.ds(tbase + (2 * p) * cn, cn)],
                    nbuf0.at[pl.ds(0, cn)], nsem0)
                d1 = pltpu.async_copy(
                    tab.at[pl.ds(tbase + (2 * p + 1) * cn, cn)],
                    nbuf1.at[pl.ds(0, cn)], nsem1)
                d0.wait()
                accs = reduce_buf(nbuf0, accs)
                d1.wait()
                accs = reduce_buf(nbuf1, accs)
                return accs

            accs = lax.fori_loop(0, nch // 2, pair, zero4)
            if nch % 2:
                pltpu.sync_copy(tab.at[pl.ds(tbase + (nch - 1) * cn, cn)],
                                nbuf0.at[pl.ds(0, cn)])
                accs = reduce_buf(nbuf0, accs)
            nacc_v[...] = accs[0] + accs[1] + accs[2] + accs[3]
            pltpu.sync_copy(nacc_v, out_row)

        table_ssq(wrd_h, o_wssq.at[wid], 125)
        table_ssq(ent_h, o_essq.at[wid], NCH)

    return k(entity_emb, word_emb)


def _log_sigmoid(x):
    return jnp.minimum(x, 0.0) - jnp.log1p(jnp.exp(-jnp.abs(x)))


def _loss_partials(user_e, item_e, qsum, nie, w_e, nw_e, wb16, nwb16,
                   rwmod, nrwmod, WqT, bq, pf):
    G = 32
    S = B // G

    def body(u_ref, it_ref, q_ref, nie_ref, we_ref, nwe_ref, wb_ref, nwb_ref,
             rwm_ref, nrwm_ref, wqt_ref, bq_ref, pf_ref,
             s1_ref, s2_ref, ns_ref, nw_ref, wbo_ref):
        i = pl.program_id(0)
        u = u_ref[...]
        it = it_ref[...]
        qmean = q_ref[...] * (1.0 / QL)
        q = jnp.tanh(jnp.dot(qmean, wqt_ref[...],
                             preferred_element_type=jnp.float32) + bq_ref[...])
        pf = pf_ref[0, 0]
        pm = pf * q + (1.0 - pf) * u

        s1p = jnp.sum(it * pm)
        nid = jnp.sum(nie_ref[...].reshape(S, K, D) * pm[:, None, :], axis=2)
        nsp = jnp.sum(-_log_sigmoid(-nid))

        s2p = jnp.sum(we_ref[...] * it)
        nwd = jnp.sum(nwe_ref[...].reshape(S, K, D) * it[:, None, :], axis=2)

        lanes = lax.broadcasted_iota(jnp.int32, (S * K, 16), 1)
        nwb = jnp.sum(jnp.where(lanes == nrwm_ref[0, 0, :][:, None],
                                nwb_ref[...], 0.0), axis=1).reshape(S, K)
        nwp = jnp.sum(-_log_sigmoid(-nwd - nwb))

        lanes2 = lax.broadcasted_iota(jnp.int32, (S, 16), 1)
        wb = jnp.sum(jnp.where(lanes2 == rwm_ref[0, 0, :][:, None],
                               wb_ref[...], 0.0), axis=1)
        wbo_ref[...] = wb.reshape(1, 1, S)

        @pl.when(i == 0)
        def _():
            s1_ref[...] = jnp.zeros_like(s1_ref)
            s2_ref[...] = jnp.zeros_like(s2_ref)
            ns_ref[...] = jnp.zeros_like(ns_ref)
            nw_ref[...] = jnp.zeros_like(nw_ref)

        s1_ref[...] += s1p.reshape(1, 1)
        s2_ref[...] += s2p.reshape(1, 1)
        ns_ref[...] += nsp.reshape(1, 1)
        nw_ref[...] += nwp.reshape(1, 1)

    return pl.pallas_call(
        body,
        grid=(G,),
        in_specs=[
            pl.BlockSpec((S, D), lambda i: (i, 0)),
            pl.BlockSpec((S, D), lambda i: (i, 0)),
            pl.BlockSpec((S, D), lambda i: (i, 0)),
            pl.BlockSpec((S * K, D), lambda i: (i, 0)),
            pl.BlockSpec((S, D), lambda i: (i, 0)),
            pl.BlockSpec((S * K, D), lambda i: (i, 0)),
            pl.BlockSpec((S, 16), lambda i: (i, 0)),
            pl.BlockSpec((S * K, 16), lambda i: (i, 0)),
            pl.BlockSpec((1, 1, S), lambda i: (i, 0, 0)),
            pl.BlockSpec((1, 1, S * K), lambda i: (i, 0, 0)),
            pl.BlockSpec((D, D), lambda i: (0, 0)),
            pl.BlockSpec((1, D), lambda i: (0, 0)),
            pl.BlockSpec((1, 1), lambda i: (0, 0)),
        ],
        out_specs=[
            pl.BlockSpec((1, 1), lambda i: (0, 0)),
            pl.BlockSpec((1, 1), lambda i: (0, 0)),
            pl.BlockSpec((1, 1), lambda i: (0, 0)),
            pl.BlockSpec((1, 1), lambda i: (0, 0)),
            pl.BlockSpec((1, 1, S), lambda i: (i, 0, 0)),
        ],
        out_shape=[
            jax.ShapeDtypeStruct((1, 1), jnp.float32),
            jax.ShapeDtypeStruct((1, 1), jnp.float32),
            jax.ShapeDtypeStruct((1, 1), jnp.float32),
            jax.ShapeDtypeStruct((1, 1), jnp.float32),
            jax.ShapeDtypeStruct((G, 1, S), jnp.float32),
        ],
    )(user_e, item_e, qsum, nie, w_e, nw_e, wb16, nwb16, rwmod, nrwmod,
      WqT, bq, pf)


def _final_combine(s1, s2, ns, nw, wssq, essq, wb2d):
    def body(s1_ref, s2_ref, ns_ref, nw_ref, ssw_ref, sse_ref, wb_ref, o_ref):
        s2 = s2_ref[0, 0]
        pos_mean = jnp.mean(-_log_sigmoid(s2 + wb_ref[...]))
        search = -_log_sigmoid(s1_ref[0, 0]) + ns_ref[0, 0]
        reg = L2 * (jnp.sqrt(jnp.sum(ssw_ref[...])) +
                    jnp.sqrt(jnp.sum(sse_ref[...])))
        o_ref[...] = (pos_mean + nw_ref[0, 0] / B + search + reg).reshape(1, 1)

    return pl.pallas_call(
        body,
        out_shape=jax.ShapeDtypeStruct((1, 1), jnp.float32),
    )(s1, s2, ns, nw, wssq, essq, wb2d)


def kernel(users, items, query_words, review_words, neg_items,
           neg_review_words, word_emb, word_bias, entity_emb, Wq, bq, pf):
    users = users.astype(jnp.int32)
    items = items.astype(jnp.int32)
    qwf = query_words.astype(jnp.int32).reshape(-1)
    rw = review_words.astype(jnp.int32)
    nrw = neg_review_words.astype(jnp.int32).reshape(-1)
    negi = neg_items.astype(jnp.int32).reshape(-1)

    bias16 = word_bias.reshape(W_NUM // 16, 16)
    rwb = rw // 16
    nrwb = nrw // 16
    rwmod = (rw % 16).reshape(32, 1, B // 32)
    nrwmod = (nrw % 16).reshape(32, 1, (B * K) // 32)

    (user_e, item_e, nie, qsum, w_e, nw_e, wb16, nwb16) = (
        _sc_gather_all(entity_emb, word_emb, bias16, users, items, negi,
                       qwf, rw, nrw, rwb, nrwb))
    wssq, essq = _sc_norms(entity_emb, word_emb)

    s1, s2, ns, nw, wbo = _loss_partials(
        user_e, item_e, qsum, nie, w_e, nw_e, wb16, nwb16, rwmod, nrwmod,
        Wq.T, bq.reshape(1, D), pf.reshape(1, 1))

    out = _final_combine(s1, s2, ns, nw, wssq, essq, wbo.reshape(128, 128))
    return out.reshape(())
